# baseline (speedup 1.0000x reference)
import functools

import jax
import jax.numpy as jnp
from jax.experimental import pallas as pl
from jax.experimental.pallas import tpu as pltpu

B = 2048
D = 1024
H = 1024
OUT = 1024
E = 4
K = 2
MAX_STEPS = 10
EPS = 1e-5
HALT_LIMIT = 1.0 - 0.01

BT = 256
NT = B // BT
SUB = 8


def _dot_t(a, b):
    return jax.lax.dot_general(
        a, b, (((1,), (1,)), ((), ())), preferred_element_type=jnp.float32)


def _xwih_body(x_ref, w_ref, b_ref, o_ref):
    o_ref[...] = _dot_t(x_ref[...], w_ref[...]) + b_ref[...]


def _act_body(xwih_ref, whh_ref, bhh_ref, hw_ref, hb_ref, o_ref,
              h_s, c_s, hsum_s, hp_s, rem_s, stat_s, dma_sem):
    s = pl.program_id(0)
    t = pl.program_id(1)
    row = pl.ds(t * BT, BT)

    @pl.when(jnp.logical_and(s == 0, t == 0))
    def _init_global():
        stat_s[0] = 1.0
        stat_s[1] = 0.0
        stat_s[2] = jnp.inf

    @pl.when(jnp.logical_and(s > 0, t == 0))
    def _update_active():
        still = jnp.where(stat_s[2] <= HALT_LIMIT, stat_s[0], 0.0)
        stat_s[0] = still
        stat_s[2] = jnp.inf

    @pl.when(t == 0)
    def _count_step():
        stat_s[1] = stat_s[1] + stat_s[0]

    @pl.when(s == 0)
    def _init_tile():
        zeros = jnp.zeros((BT, H), jnp.float32)
        h_s[row, :] = zeros
        c_s[row, :] = zeros
        hsum_s[row, :] = zeros
        hp_s[row, :] = jnp.zeros((BT, 1), jnp.float32)
        rem_s[row, :] = jnp.zeros((BT, 1), jnp.float32)

    active = stat_s[0] > 0.5

    @pl.when(active)
    def _step():
        h = h_s[row, :]
        c = c_s[row, :]
        def gate(k):
            return ((xwih_ref[:, k * H:(k + 1) * H]
                     + _dot_t(h, whh_ref[k * H:(k + 1) * H, :]))
                    + bhh_ref[:, k * H:(k + 1) * H])
        i = jax.nn.sigmoid(gate(0))
        f = jax.nn.sigmoid(gate(1))
        g = jnp.tanh(gate(2))
        o = jax.nn.sigmoid(gate(3))
        c_new = f * c + i * g
        h_new = o * jnp.tanh(c_new)
        y = jax.nn.sigmoid(
            _dot_t(h_new, hw_ref[...])[:, 0:1] + hb_ref[0, 0])
        hp = hp_s[row, :]
        hp_new = hp + y * (1.0 - hp)
        hp_s[row, :] = hp_new
        rem_s[row, :] = rem_s[row, :] + (1.0 - hp_new)
        h_s[row, :] = h_new
        c_s[row, :] = c_new
        hsum_s[row, :] = hsum_s[row, :] + h_new
        stat_s[2] = jnp.minimum(stat_s[2], jnp.min(hp_new))

    @pl.when(s == MAX_STEPS - 1)
    def _emit():
        hsum_s[row, :] = rem_s[row, :] * hsum_s[row, :] / stat_s[1]
        copy = pltpu.make_async_copy(
            hsum_s.at[row, :], o_ref.at[row, :], dma_sem)
        copy.start()
        copy.wait()


def _moe_layer(x, e, t, gw_ref, gb_ref, w1_ref, b1_ref, w2_ref, b2_ref,
               acc_s, wfull_s):
    row = pl.ds(SUB * t * BT, SUB * BT)

    @pl.when(e == 0)
    def _gate():
        logits = _dot_t(x, gw_ref[...]) + gb_ref[...]
        col = jax.lax.broadcasted_iota(jnp.int32, (SUB * BT, E), 1)
        ranks = jnp.zeros((SUB * BT, E), jnp.float32)
        for j in range(E):
            lj = logits[:, j:j + 1]
            beats = (lj > logits) | ((lj == logits) & (col > j))
            ranks = ranks + beats.astype(jnp.float32)
        keep = ranks < K
        m = jnp.max(logits, axis=1, keepdims=True)
        ex = jnp.exp(logits - m) * keep.astype(jnp.float32)
        wfull_s[row, :] = ex / jnp.sum(ex, axis=1, keepdims=True)

    col = jax.lax.broadcasted_iota(jnp.int32, (BT, E), 1)
    for sub in range(SUB):
        srow = pl.ds((SUB * t + sub) * BT, BT)
        xs = x[sub * BT:(sub + 1) * BT, :]
        hidden = jnp.maximum(_dot_t(xs, w1_ref[0]) + b1_ref[0], 0.0)
        out_e = _dot_t(hidden, w2_ref[0]) + b2_ref[0]
        w_e = jnp.sum(wfull_s[srow, :] * (col == e).astype(jnp.float32),
                      axis=1, keepdims=True)
        contrib = w_e * out_e

        @pl.when(e == 0)
        def _acc_init(srow=srow, contrib=contrib):
            acc_s[srow, :] = contrib

        @pl.when(e > 0)
        def _acc_add(srow=srow, contrib=contrib):
            acc_s[srow, :] = acc_s[srow, :] + contrib


def _moe_body(apply_ln, xin_ref, gw_ref, gb_ref, w1_ref, b1_ref,
              w2_ref, b2_ref, gamma_ref, beta_ref, o_ref,
              acc_s, wfull_s, dma_sem):
    e = pl.program_id(0)
    t = pl.program_id(1)
    row = pl.ds(SUB * t * BT, SUB * BT)

    _moe_layer(xin_ref[...], e, t, gw_ref, gb_ref, w1_ref, b1_ref,
               w2_ref, b2_ref, acc_s, wfull_s)

    @pl.when(e == E - 1)
    def _emit():
        if apply_ln:
            acc = acc_s[row, :]
            mu = jnp.mean(acc, axis=-1, keepdims=True)
            xc = acc - mu
            var = jnp.mean(xc * xc, axis=-1, keepdims=True)
            acc_s[row, :] = (xc / jnp.sqrt(var + EPS) * gamma_ref[...]
                             + beta_ref[...])
        copy = pltpu.make_async_copy(
            acc_s.at[row, :], o_ref.at[row, :], dma_sem)
        copy.start()
        copy.wait()


def _moe_call(xin, gate_W, gate_b, fc1_W, fc1_b, fc2_W, fc2_b,
              gamma, beta, apply_ln):
    n_out = fc2_W.shape[1]
    return pl.pallas_call(
        functools.partial(_moe_body, apply_ln),
        grid=(E, NT // SUB),
        in_specs=[
            pl.BlockSpec((SUB * BT, H), lambda e, t: (t, 0)),
            pl.BlockSpec((E, H), lambda e, t: (0, 0)),
            pl.BlockSpec((1, E), lambda e, t: (0, 0)),
            pl.BlockSpec((1, H, H), lambda e, t: (e, 0, 0)),
            pl.BlockSpec((1, 1, H), lambda e, t: (e, 0, 0)),
            pl.BlockSpec((1, n_out, H), lambda e, t: (e, 0, 0)),
            pl.BlockSpec((1, 1, n_out), lambda e, t: (e, 0, 0)),
            pl.BlockSpec((1, n_out), lambda e, t: (0, 0)),
            pl.BlockSpec((1, n_out), lambda e, t: (0, 0)),
        ],
        out_specs=pl.BlockSpec(memory_space=pl.ANY),
        out_shape=jax.ShapeDtypeStruct((B, n_out), jnp.float32),
        scratch_shapes=[
            pltpu.VMEM((B, n_out), jnp.float32),
            pltpu.VMEM((B, E), jnp.float32),
            pltpu.SemaphoreType.DMA,
        ],
        compiler_params=pltpu.CompilerParams(
            dimension_semantics=("arbitrary", "arbitrary")),
    )(xin, gate_W, gate_b.reshape(1, E), fc1_W, fc1_b[:, None, :],
      fc2_W, fc2_b[:, None, :],
      gamma.reshape(1, n_out), beta.reshape(1, n_out))


def kernel(x, lstm_Wih, lstm_Whh, lstm_bih, lstm_bhh, halt_W, halt_b,
           moe0_gate_W, moe0_gate_b, moe0_fc1_W, moe0_fc1_b,
           moe0_fc2_W, moe0_fc2_b,
           moe1_gate_W, moe1_gate_b, moe1_fc1_W, moe1_fc1_b,
           moe1_fc2_W, moe1_fc2_b, ln_gamma, ln_beta):
    bias = lstm_bih.reshape(1, 4 * H)

    xwih = pl.pallas_call(
        _xwih_body,
        grid=(NT,),
        in_specs=[
            pl.BlockSpec((BT, D), lambda t: (t, 0)),
            pl.BlockSpec((4 * H, D), lambda t: (0, 0)),
            pl.BlockSpec((1, 4 * H), lambda t: (0, 0)),
        ],
        out_specs=pl.BlockSpec((BT, 4 * H), lambda t: (t, 0)),
        out_shape=jax.ShapeDtypeStruct((B, 4 * H), jnp.float32),
        compiler_params=pltpu.CompilerParams(
            dimension_semantics=("arbitrary",)),
    )(x, lstm_Wih, bias)

    a = pl.pallas_call(
        _act_body,
        grid=(MAX_STEPS, NT),
        in_specs=[
            pl.BlockSpec((BT, 4 * H), lambda s, t: (t, 0)),
            pl.BlockSpec((4 * H, H), lambda s, t: (0, 0)),
            pl.BlockSpec((1, 4 * H), lambda s, t: (0, 0)),
            pl.BlockSpec((128, H), lambda s, t: (0, 0)),
            pl.BlockSpec((1, 1), lambda s, t: (0, 0)),
        ],
        out_specs=pl.BlockSpec(memory_space=pl.ANY),
        out_shape=jax.ShapeDtypeStruct((B, H), jnp.float32),
        scratch_shapes=[
            pltpu.VMEM((B, H), jnp.float32),
            pltpu.VMEM((B, H), jnp.float32),
            pltpu.VMEM((B, H), jnp.float32),
            pltpu.VMEM((B, 1), jnp.float32),
            pltpu.VMEM((B, 1), jnp.float32),
            pltpu.SMEM((3,), jnp.float32),
            pltpu.SemaphoreType.DMA,
        ],
        compiler_params=pltpu.CompilerParams(
            dimension_semantics=("arbitrary", "arbitrary")),
    )(xwih, lstm_Whh, lstm_bhh.reshape(1, 4 * H),
      jnp.zeros((128, H), jnp.float32).at[0].set(halt_W[0]),
      halt_b.reshape(1, 1))

    h1 = _moe_call(a, moe0_gate_W, moe0_gate_b, moe0_fc1_W, moe0_fc1_b,
                   moe0_fc2_W, moe0_fc2_b, ln_gamma, ln_beta, apply_ln=False)
    return _moe_call(h1, moe1_gate_W, moe1_gate_b, moe1_fc1_W, moe1_fc1_b,
                     moe1_fc2_W, moe1_fc2_b, ln_gamma, ln_beta, apply_ln=True)


# speedup vs baseline: 1.0304x; 1.0304x over previous
import functools

import jax
import jax.numpy as jnp
from jax.experimental import pallas as pl
from jax.experimental.pallas import tpu as pltpu

B = 2048
D = 1024
H = 1024
OUT = 1024
E = 4
K = 2
MAX_STEPS = 10
EPS = 1e-5
HALT_LIMIT = 1.0 - 0.01

BT = 256
NT = B // BT
SUB = 4


def _dot_t(a, b):
    return jax.lax.dot_general(
        a, b, (((1,), (1,)), ((), ())), preferred_element_type=jnp.float32)


def _xwih_body(x_ref, w_ref, b_ref, o_ref):
    o_ref[...] = _dot_t(x_ref[...], w_ref[...]) + b_ref[...]


def _act_body(xwih_ref, whh_ref, bhh_ref, hw_ref, hb_ref, o_ref,
              h_s, c_s, hsum_s, hp_s, rem_s, stat_s, dma_sem):
    s = pl.program_id(0)
    t = pl.program_id(1)
    row = pl.ds(t * BT, BT)

    @pl.when(jnp.logical_and(s == 0, t == 0))
    def _init_global():
        stat_s[0] = 1.0
        stat_s[1] = 0.0
        stat_s[2] = jnp.inf

    @pl.when(jnp.logical_and(s > 0, t == 0))
    def _update_active():
        still = jnp.where(stat_s[2] <= HALT_LIMIT, stat_s[0], 0.0)
        stat_s[0] = still
        stat_s[2] = jnp.inf

    @pl.when(t == 0)
    def _count_step():
        stat_s[1] = stat_s[1] + stat_s[0]

    @pl.when(s == 0)
    def _init_tile():
        zeros = jnp.zeros((BT, H), jnp.float32)
        h_s[row, :] = zeros
        c_s[row, :] = zeros
        hsum_s[row, :] = zeros
        hp_s[row, :] = jnp.zeros((BT, 1), jnp.float32)
        rem_s[row, :] = jnp.zeros((BT, 1), jnp.float32)

    active = stat_s[0] > 0.5

    @pl.when(active)
    def _step():
        h = h_s[row, :]
        c = c_s[row, :]
        def gate(k):
            return ((xwih_ref[:, k * H:(k + 1) * H]
                     + _dot_t(h, whh_ref[k * H:(k + 1) * H, :]))
                    + bhh_ref[:, k * H:(k + 1) * H])
        i = jax.nn.sigmoid(gate(0))
        f = jax.nn.sigmoid(gate(1))
        g = jnp.tanh(gate(2))
        o = jax.nn.sigmoid(gate(3))
        c_new = f * c + i * g
        h_new = o * jnp.tanh(c_new)
        y = jax.nn.sigmoid(
            _dot_t(h_new, hw_ref[...])[:, 0:1] + hb_ref[0, 0])
        hp = hp_s[row, :]
        hp_new = hp + y * (1.0 - hp)
        hp_s[row, :] = hp_new
        rem_s[row, :] = rem_s[row, :] + (1.0 - hp_new)
        h_s[row, :] = h_new
        c_s[row, :] = c_new
        hsum_s[row, :] = hsum_s[row, :] + h_new
        stat_s[2] = jnp.minimum(stat_s[2], jnp.min(hp_new))

    @pl.when(s == MAX_STEPS - 1)
    def _emit():
        hsum_s[row, :] = rem_s[row, :] * hsum_s[row, :] / stat_s[1]
        copy = pltpu.make_async_copy(
            hsum_s.at[row, :], o_ref.at[row, :], dma_sem)
        copy.start()
        copy.wait()


def _moe_layer(x, e, t, gw_ref, gb_ref, w1_ref, b1_ref, w2_ref, b2_ref,
               acc_s, wfull_s):
    row = pl.ds(SUB * t * BT, SUB * BT)

    @pl.when(e == 0)
    def _gate():
        logits = _dot_t(x, gw_ref[...]) + gb_ref[...]
        col = jax.lax.broadcasted_iota(jnp.int32, (SUB * BT, E), 1)
        ranks = jnp.zeros((SUB * BT, E), jnp.float32)
        for j in range(E):
            lj = logits[:, j:j + 1]
            beats = (lj > logits) | ((lj == logits) & (col > j))
            ranks = ranks + beats.astype(jnp.float32)
        keep = ranks < K
        m = jnp.max(logits, axis=1, keepdims=True)
        ex = jnp.exp(logits - m) * keep.astype(jnp.float32)
        wfull_s[row, :] = ex / jnp.sum(ex, axis=1, keepdims=True)

    col = jax.lax.broadcasted_iota(jnp.int32, (BT, E), 1)
    for sub in range(SUB):
        srow = pl.ds((SUB * t + sub) * BT, BT)
        xs = x[sub * BT:(sub + 1) * BT, :]
        hidden = jnp.maximum(_dot_t(xs, w1_ref[0]) + b1_ref[0], 0.0)
        out_e = _dot_t(hidden, w2_ref[0]) + b2_ref[0]
        w_e = jnp.sum(wfull_s[srow, :] * (col == e).astype(jnp.float32),
                      axis=1, keepdims=True)
        contrib = w_e * out_e

        @pl.when(e == 0)
        def _acc_init(srow=srow, contrib=contrib):
            acc_s[srow, :] = contrib

        @pl.when(e > 0)
        def _acc_add(srow=srow, contrib=contrib):
            acc_s[srow, :] = acc_s[srow, :] + contrib


def _moe_body(apply_ln, xin_ref, gw_ref, gb_ref, w1_ref, b1_ref,
              w2_ref, b2_ref, gamma_ref, beta_ref, o_ref,
              acc_s, wfull_s, dma_sem):
    e = pl.program_id(0)
    t = pl.program_id(1)
    row = pl.ds(SUB * t * BT, SUB * BT)

    _moe_layer(xin_ref[...], e, t, gw_ref, gb_ref, w1_ref, b1_ref,
               w2_ref, b2_ref, acc_s, wfull_s)

    @pl.when(e == E - 1)
    def _emit():
        if apply_ln:
            acc = acc_s[row, :]
            mu = jnp.mean(acc, axis=-1, keepdims=True)
            xc = acc - mu
            var = jnp.mean(xc * xc, axis=-1, keepdims=True)
            acc_s[row, :] = (xc / jnp.sqrt(var + EPS) * gamma_ref[...]
                             + beta_ref[...])
        copy = pltpu.make_async_copy(
            acc_s.at[row, :], o_ref.at[row, :], dma_sem)
        copy.start()
        copy.wait()


def _moe_call(xin, gate_W, gate_b, fc1_W, fc1_b, fc2_W, fc2_b,
              gamma, beta, apply_ln):
    n_out = fc2_W.shape[1]
    return pl.pallas_call(
        functools.partial(_moe_body, apply_ln),
        grid=(E, NT // SUB),
        in_specs=[
            pl.BlockSpec((SUB * BT, H), lambda e, t: (t, 0)),
            pl.BlockSpec((E, H), lambda e, t: (0, 0)),
            pl.BlockSpec((1, E), lambda e, t: (0, 0)),
            pl.BlockSpec((1, H, H), lambda e, t: (e, 0, 0)),
            pl.BlockSpec((1, 1, H), lambda e, t: (e, 0, 0)),
            pl.BlockSpec((1, n_out, H), lambda e, t: (e, 0, 0)),
            pl.BlockSpec((1, 1, n_out), lambda e, t: (e, 0, 0)),
            pl.BlockSpec((1, n_out), lambda e, t: (0, 0)),
            pl.BlockSpec((1, n_out), lambda e, t: (0, 0)),
        ],
        out_specs=pl.BlockSpec(memory_space=pl.ANY),
        out_shape=jax.ShapeDtypeStruct((B, n_out), jnp.float32),
        scratch_shapes=[
            pltpu.VMEM((B, n_out), jnp.float32),
            pltpu.VMEM((B, E), jnp.float32),
            pltpu.SemaphoreType.DMA,
        ],
        compiler_params=pltpu.CompilerParams(
            dimension_semantics=("arbitrary", "arbitrary")),
    )(xin, gate_W, gate_b.reshape(1, E), fc1_W, fc1_b[:, None, :],
      fc2_W, fc2_b[:, None, :],
      gamma.reshape(1, n_out), beta.reshape(1, n_out))


def kernel(x, lstm_Wih, lstm_Whh, lstm_bih, lstm_bhh, halt_W, halt_b,
           moe0_gate_W, moe0_gate_b, moe0_fc1_W, moe0_fc1_b,
           moe0_fc2_W, moe0_fc2_b,
           moe1_gate_W, moe1_gate_b, moe1_fc1_W, moe1_fc1_b,
           moe1_fc2_W, moe1_fc2_b, ln_gamma, ln_beta):
    bias = lstm_bih.reshape(1, 4 * H)

    xwih = pl.pallas_call(
        _xwih_body,
        grid=(NT,),
        in_specs=[
            pl.BlockSpec((BT, D), lambda t: (t, 0)),
            pl.BlockSpec((4 * H, D), lambda t: (0, 0)),
            pl.BlockSpec((1, 4 * H), lambda t: (0, 0)),
        ],
        out_specs=pl.BlockSpec((BT, 4 * H), lambda t: (t, 0)),
        out_shape=jax.ShapeDtypeStruct((B, 4 * H), jnp.float32),
        compiler_params=pltpu.CompilerParams(
            dimension_semantics=("arbitrary",)),
    )(x, lstm_Wih, bias)

    a = pl.pallas_call(
        _act_body,
        grid=(MAX_STEPS, NT),
        in_specs=[
            pl.BlockSpec((BT, 4 * H), lambda s, t: (t, 0)),
            pl.BlockSpec((4 * H, H), lambda s, t: (0, 0)),
            pl.BlockSpec((1, 4 * H), lambda s, t: (0, 0)),
            pl.BlockSpec((128, H), lambda s, t: (0, 0)),
            pl.BlockSpec((1, 1), lambda s, t: (0, 0)),
        ],
        out_specs=pl.BlockSpec(memory_space=pl.ANY),
        out_shape=jax.ShapeDtypeStruct((B, H), jnp.float32),
        scratch_shapes=[
            pltpu.VMEM((B, H), jnp.float32),
            pltpu.VMEM((B, H), jnp.float32),
            pltpu.VMEM((B, H), jnp.float32),
            pltpu.VMEM((B, 1), jnp.float32),
            pltpu.VMEM((B, 1), jnp.float32),
            pltpu.SMEM((3,), jnp.float32),
            pltpu.SemaphoreType.DMA,
        ],
        compiler_params=pltpu.CompilerParams(
            dimension_semantics=("arbitrary", "arbitrary")),
    )(xwih, lstm_Whh, lstm_bhh.reshape(1, 4 * H),
      jnp.zeros((128, H), jnp.float32).at[0].set(halt_W[0]),
      halt_b.reshape(1, 1))

    h1 = _moe_call(a, moe0_gate_W, moe0_gate_b, moe0_fc1_W, moe0_fc1_b,
                   moe0_fc2_W, moe0_fc2_b, ln_gamma, ln_beta, apply_ln=False)
    return _moe_call(h1, moe1_gate_W, moe1_gate_b, moe1_fc1_W, moe1_fc1_b,
                     moe1_fc2_W, moe1_fc2_b, ln_gamma, ln_beta, apply_ln=True)
